# baseline (speedup 1.0000x reference)
import functools

import jax
import jax.numpy as jnp
from jax import lax
from jax.experimental import pallas as pl
from jax.experimental.pallas import tpu as pltpu
from jax.experimental.pallas import tpu_sc as plsc

VOCAB = 1000000
D = 64
T = 200
B = 4096

NC = 2
NS = 16
NW = NC * NS

CH = B // NW
NBUF = 4

_mesh = plsc.VectorSubcoreMesh(core_axis_name="c", subcore_axis_name="s")

_scratch = []
for _ in range(NBUF):
    _scratch += [
        pltpu.VMEM((CH,), jnp.int32),
        pltpu.VMEM((CH, 2 * D), jnp.float32),
        pltpu.VMEM((D, CH), jnp.float32),
    ]
_scratch += [
    pltpu.VMEM((T * D,), jnp.float32),
    pltpu.SemaphoreType.DMA((NBUF,)),
    pltpu.SemaphoreType.DMA((NBUF,)),
    pltpu.SemaphoreType.DMA((NBUF,)),
]


@functools.partial(
    pl.kernel,
    mesh=_mesh,
    out_type=jax.ShapeDtypeStruct((T, D, B), jnp.float32),
    compiler_params=pltpu.CompilerParams(needs_layout_passes=False),
    scratch_types=_scratch,
)
def _embed(xt_hbm, tab_hbm, pos_hbm, out_hbm, *scr):
    xi = [scr[3 * b + 0] for b in range(NBUF)]
    rv = [scr[3 * b + 1] for b in range(NBUF)]
    tv = [scr[3 * b + 2] for b in range(NBUF)]
    pos_v, sem_i, sem_g, sem_o = scr[3 * NBUF:]

    wid = lax.axis_index("s") * NC + lax.axis_index("c")
    col = pl.multiple_of(wid * CH, CH)
    pltpu.sync_copy(pos_hbm, pos_v)
    lane = lax.iota(jnp.int32, 16)

    def group_body(g):
        for b in range(NBUF):
            t = g + b
            pltpu.async_copy(xt_hbm.at[t, pl.ds(col, CH)], xi[b],
                             sem_i.at[b])
        for b in range(NBUF):
            t = g + b
            pltpu.make_async_copy(xt_hbm.at[t, pl.ds(col, CH)], xi[b],
                                  sem_i.at[b]).wait()
            pltpu.async_copy(tab_hbm.at[xi[b]], rv[b], sem_g.at[b])
        for b in range(NBUF):
            t = g + b
            pltpu.make_async_copy(tab_hbm.at[xi[b]], rv[b],
                                  sem_g.at[b]).wait()
            posv = [pos_v[pl.ds(t * D + fg * 16, 16)] for fg in range(D // 16)]
            rowidx = [lane + fg * 16 for fg in range(D // 16)]

            def row_body(i, carry, b=b, posv=posv, rowidx=rowidx):
                colv = jnp.full((16,), 0, jnp.int32) + i
                for fg in range(D // 16):
                    val = rv[b][i, pl.ds(fg * 16, 16)] + posv[fg]
                    plsc.store_scatter(tv[b], [rowidx[fg], colv], val)
                return carry

            lax.fori_loop(0, CH, row_body, 0, unroll=4)
            pltpu.async_copy(tv[b], out_hbm.at[t, :, pl.ds(col, CH)],
                             sem_o.at[b])
        for b in range(NBUF):
            t = g + b
            pltpu.make_async_copy(tv[b], out_hbm.at[t, :, pl.ds(col, CH)],
                                  sem_o.at[b]).wait()

    pl.loop(0, T, step=NBUF)(group_body)


def kernel(x, text_embedding, positional_embedding):
    xt = x.T.astype(jnp.int32)
    tab = jnp.pad(text_embedding, ((0, 0), (0, D)))
    pflat = positional_embedding.reshape(-1)
    out = _embed(xt, tab, pflat)
    return out.transpose(2, 0, 1)


# speedup vs baseline: 1.2422x; 1.2422x over previous
import functools

import jax
import jax.numpy as jnp
from jax import lax
from jax.experimental import pallas as pl
from jax.experimental.pallas import tpu as pltpu
from jax.experimental.pallas import tpu_sc as plsc

VOCAB = 1000000
D = 64
T = 200
B = 4096

NC = 2
NS = 16
NW = NC * NS

ROWS = B // NW
NBUF = 2

_mesh = plsc.VectorSubcoreMesh(core_axis_name="c", subcore_axis_name="s")

_scratch = []
for _ in range(NBUF):
    _scratch += [
        pltpu.VMEM((T,), jnp.int32),
        pltpu.VMEM((T, 2 * D), jnp.float32),
        pltpu.VMEM((T * D,), jnp.float32),
    ]
_scratch += [
    pltpu.VMEM((T, D), jnp.float32),
    pltpu.SemaphoreType.DMA((NBUF,)),
    pltpu.SemaphoreType.DMA((NBUF,)),
    pltpu.SemaphoreType.DMA((NBUF,)),
]


@functools.partial(
    pl.kernel,
    mesh=_mesh,
    out_type=jax.ShapeDtypeStruct((B, T * D), jnp.float32),
    compiler_params=pltpu.CompilerParams(needs_layout_passes=False),
    scratch_types=_scratch,
)
def _embed(x_hbm, tab_hbm, pos_hbm, out_hbm, *scr):
    xi = [scr[3 * b + 0] for b in range(NBUF)]
    rv = [scr[3 * b + 1] for b in range(NBUF)]
    ov = [scr[3 * b + 2] for b in range(NBUF)]
    pos_v, sem_i, sem_g, sem_o = scr[3 * NBUF:]

    wid = lax.axis_index("s") * NC + lax.axis_index("c")
    row0 = pl.multiple_of(wid * ROWS, ROWS)
    pltpu.sync_copy(pos_hbm, pos_v)

    def group_body(g):
        for b in range(NBUF):
            pltpu.async_copy(x_hbm.at[row0 + g + b], xi[b], sem_i.at[b])
        for b in range(NBUF):
            pltpu.make_async_copy(x_hbm.at[row0 + g + b], xi[b],
                                  sem_i.at[b]).wait()
            pltpu.async_copy(tab_hbm.at[xi[b]], rv[b], sem_g.at[b])
        for b in range(NBUF):
            pltpu.make_async_copy(tab_hbm.at[xi[b]], rv[b],
                                  sem_g.at[b]).wait()

            def tok_body(i, carry, b=b):
                for fg in range(D // 16):
                    sl = pl.ds(fg * 16, 16)
                    ov[b][pl.ds(i * D + fg * 16, 16)] = (
                        rv[b][i, sl] + pos_v[i, sl])
                return carry

            lax.fori_loop(0, T, tok_body, 0, unroll=8)
            pltpu.async_copy(ov[b], out_hbm.at[row0 + g + b], sem_o.at[b])
        for b in range(NBUF):
            pltpu.make_async_copy(ov[b], out_hbm.at[row0 + g + b],
                                  sem_o.at[b]).wait()

    pl.loop(0, ROWS, step=NBUF)(group_body)


def kernel(x, text_embedding, positional_embedding):
    tab = jnp.pad(text_embedding, ((0, 0), (0, D)))
    out = _embed(x.astype(jnp.int32), tab, positional_embedding)
    return out.reshape(B, T, D)
